# Initial kernel scaffold; baseline (speedup 1.0000x reference)
import jax
import jax.numpy as jnp
from jax.experimental import pallas as pl


def kernel(inputs, router_w, expert_w, expert_b):
    raise NotImplementedError("write your pallas kernel here")


# speedup vs baseline: 3.8537x; 3.8537x over previous
import functools

import jax
import jax.numpy as jnp
from jax.experimental import pallas as pl

NUM_EXPERTS = 64
MAX_GROUP_SIZE = 4096
CAPACITY_FACTOR = 1.0
MIN_EXPERT_CAPACITY = 4


def _num_groups(num_tokens, max_group_size, num_experts):
    n = max(num_tokens // max_group_size, num_experts)
    while n < num_tokens and not (num_tokens % n == 0 and n % num_experts == 0):
        n += 1
    return n


def _moe_kernel(x_ref, rw_ref, w_ref, b_ref, out_ref, *, tpg, capacity):
    x = x_ref[0]
    logits = jnp.dot(x, rw_ref[...], preferred_element_type=jnp.float32)
    m = jnp.max(logits, axis=1, keepdims=True)
    p = jnp.exp(logits - m)
    probs = p / jnp.sum(p, axis=1, keepdims=True)

    tok_iota = jax.lax.broadcasted_iota(jnp.int32, probs.shape, 0)
    work = probs
    gate = jnp.zeros_like(probs)
    for _ in range(capacity):
        col_max = jnp.max(work, axis=0, keepdims=True)
        is_max = work == col_max
        sel_idx = jnp.min(jnp.where(is_max, tok_iota, tpg), axis=0, keepdims=True)
        sel = tok_iota == sel_idx
        gate = jnp.where(sel, probs, gate)
        work = jnp.where(sel, -1.0, work)
    scale = jnp.sum(gate, axis=1, keepdims=True)

    y = jnp.dot(x.astype(jnp.bfloat16), w_ref[...],
                preferred_element_type=jnp.float32)
    y = (y.astype(jnp.bfloat16) + b_ref[...]).astype(jnp.float32)
    out_ref[0] = scale * y


def kernel(inputs, router_w, expert_w, expert_b):
    b, s, d = inputs.shape
    num_tokens = b * s
    num_groups = _num_groups(num_tokens, MAX_GROUP_SIZE, NUM_EXPERTS)
    tpg = num_tokens // num_groups
    capacity = max(int(round(CAPACITY_FACTOR * tpg / NUM_EXPERTS)),
                   MIN_EXPERT_CAPACITY)
    x = inputs.reshape(num_groups, tpg, d)
    w_bf16 = expert_w.astype(jnp.bfloat16)
    b_bf16 = expert_b.astype(jnp.bfloat16).reshape(1, d)

    out = pl.pallas_call(
        functools.partial(_moe_kernel, tpg=tpg, capacity=capacity),
        grid=(num_groups,),
        in_specs=[
            pl.BlockSpec((1, tpg, d), lambda g: (g, 0, 0)),
            pl.BlockSpec((d, NUM_EXPERTS), lambda g: (0, 0)),
            pl.BlockSpec((d, d), lambda g: (0, 0)),
            pl.BlockSpec((1, d), lambda g: (0, 0)),
        ],
        out_specs=pl.BlockSpec((1, tpg, d), lambda g: (g, 0, 0)),
        out_shape=jax.ShapeDtypeStruct((num_groups, tpg, d), jnp.float32),
    )(x, router_w, w_bf16, b_bf16)
    return out.reshape(b, s, d)


# speedup vs baseline: 4.5430x; 1.1789x over previous
import functools

import jax
import jax.numpy as jnp
from jax.experimental import pallas as pl

NUM_EXPERTS = 64
MAX_GROUP_SIZE = 4096
CAPACITY_FACTOR = 1.0
MIN_EXPERT_CAPACITY = 4


def _num_groups(num_tokens, max_group_size, num_experts):
    n = max(num_tokens // max_group_size, num_experts)
    while n < num_tokens and not (num_tokens % n == 0 and n % num_experts == 0):
        n += 1
    return n


def _moe_kernel(x_ref, rwt_ref, w_ref, b_ref, out_ref, *, tpg, capacity):
    x = x_ref[0]
    logits_t = jax.lax.dot_general(
        rwt_ref[...], x, (((1,), (1,)), ((), ())),
        preferred_element_type=jnp.float32)
    m = jnp.max(logits_t, axis=0, keepdims=True)
    p = jnp.exp(logits_t - m)
    probs = p / jnp.sum(p, axis=0, keepdims=True)

    work = probs
    for _ in range(capacity - 1):
        row_max = jnp.max(work, axis=1, keepdims=True)
        work = jnp.where(work >= row_max, -1.0, work)
    tau = jnp.max(work, axis=1, keepdims=True)
    gate = jnp.where(probs >= tau, probs, 0.0)
    scale = jnp.sum(gate, axis=0, keepdims=True)

    y = jnp.dot(x.astype(jnp.bfloat16), w_ref[...],
                preferred_element_type=jnp.float32)
    out_ref[0] = scale.T * (y + b_ref[...])


def kernel(inputs, router_w, expert_w, expert_b):
    b, s, d = inputs.shape
    num_tokens = b * s
    num_groups = _num_groups(num_tokens, MAX_GROUP_SIZE, NUM_EXPERTS)
    tpg = num_tokens // num_groups
    capacity = max(int(round(CAPACITY_FACTOR * tpg / NUM_EXPERTS)),
                   MIN_EXPERT_CAPACITY)
    x = inputs.reshape(num_groups, tpg, d)
    rwt = router_w.T
    w_bf16 = expert_w.astype(jnp.bfloat16)
    b_f32 = expert_b.astype(jnp.float32).reshape(1, d)

    out = pl.pallas_call(
        functools.partial(_moe_kernel, tpg=tpg, capacity=capacity),
        grid=(num_groups,),
        in_specs=[
            pl.BlockSpec((1, tpg, d), lambda g: (g, 0, 0)),
            pl.BlockSpec((NUM_EXPERTS, d), lambda g: (0, 0)),
            pl.BlockSpec((d, d), lambda g: (0, 0)),
            pl.BlockSpec((1, d), lambda g: (0, 0)),
        ],
        out_specs=pl.BlockSpec((1, tpg, d), lambda g: (g, 0, 0)),
        out_shape=jax.ShapeDtypeStruct((num_groups, tpg, d), jnp.float32),
    )(x, rwt, w_bf16, b_f32)
    return out.reshape(b, s, d)
